# baseline (speedup 1.0000x reference)
import jax
import jax.numpy as jnp
from jax import lax
from jax.experimental import pallas as pl
from jax.experimental.pallas import tpu as pltpu
from jax.experimental.pallas import tpu_sc as plsc

N_NODES = 50000
N_REL = 4
N_EDGES = 160000
NX = 50048
EROWS = 80
EP = 16 * EROWS * 128
ND = 50176
NHIST = 8
NDS = ND // 16
NPR = 50176
HROWS = EROWS // 2

_SC_PARAMS = pltpu.CompilerParams(
    needs_layout_passes=False, use_tc_tiling_on_sc=False)


def _rsqrt16(x):
    xi = plsc.bitcast(x, jnp.int32)
    yi = jnp.int32(0x5F3759DF) - lax.shift_right_logical(xi, 1)
    y = plsc.bitcast(yi, jnp.float32)
    for _ in range(3):
        y = y * (1.5 - 0.5 * x * y * y)
    return y


def _ce_body(srcp_hbm, dstp_hbm, ewp_hbm, cep_hbm,
             src_c, dst_c, ce_c, deg_l, redbuf, sumbuf,
             stag_sp, degsrc_sp, degdst_sp):
    c = lax.axis_index("c")
    s = lax.axis_index("s")
    zeros16 = jnp.zeros((16,), jnp.float32)
    ones16 = jnp.ones((16,), jnp.float32)

    def zero_deg_l():
        def body(i, _):
            deg_l[pl.ds(16 * i, 16)] = zeros16
            return 0
        lax.fori_loop(0, ND // 16, body, 0)

    def hist_idx():
        def body(j, _):
            for k in range(8):
                v = src_c[j, pl.ds(16 * k, 16)]
                plsc.addupdate_scatter(deg_l, [v], ones16)
            return 0
        lax.fori_loop(0, EROWS, body, 0)

    def hist_phase(idx_hbm, r):
        @pl.when(s < NHIST)
        def _():
            zero_deg_l()
            for q in range(16 // NHIST):
                pltpu.sync_copy(idx_hbm.at[r, s * (16 // NHIST) + q], src_c)
                hist_idx()
            pltpu.sync_copy(deg_l, stag_sp.at[pl.ds(s * ND, ND)])

    def reduce_phase(dsp):
        pltpu.sync_copy(stag_sp.at[pl.ds(s * NDS, NDS)], sumbuf)
        for k in range(1, NHIST):
            pltpu.sync_copy(stag_sp.at[pl.ds(k * ND + s * NDS, NDS)], redbuf)

            def add_body(i, _):
                sl = pl.ds(16 * i, 16)
                sumbuf[sl] = sumbuf[sl] + redbuf[sl]
                return 0

            lax.fori_loop(0, NDS // 16, add_body, 0)
        pltpu.sync_copy(sumbuf, dsp.at[pl.ds(s * NDS, NDS)])

    def ce_pass(idx_ref):
        def body(j, _):
            for k in range(8):
                sl = pl.ds(16 * k, 16)
                v = idx_ref[j, sl]
                d = plsc.load_gather(deg_l, [v])
                y = _rsqrt16(jnp.maximum(d, 1.0))
                ce_c[j, sl] = ce_c[j, sl] * y
            return 0
        lax.fori_loop(0, EROWS, body, 0)

    for rl in range(2):
        r = c * 2 + rl
        pltpu.sync_copy(dstp_hbm.at[r, s], dst_c)
        pltpu.sync_copy(ewp_hbm.at[r, s], ce_c)

        hist_phase(srcp_hbm, r)
        plsc.subcore_barrier()
        reduce_phase(degsrc_sp)
        plsc.subcore_barrier()
        hist_phase(dstp_hbm, r)
        plsc.subcore_barrier()
        reduce_phase(degdst_sp)
        plsc.subcore_barrier()

        pltpu.sync_copy(srcp_hbm.at[r, s], src_c)
        pltpu.sync_copy(degsrc_sp, deg_l)
        ce_pass(src_c)
        pltpu.sync_copy(degdst_sp, deg_l)
        ce_pass(dst_c)
        pltpu.sync_copy(ce_c, cep_hbm.at[r, s])
        plsc.subcore_barrier()


def _agg_body(xt_hbm, srcp_hbm, dstp_hbm, cep_hbm, P_hbm,
              src_c, dst_c, ce_c, rows_a, rows_b, q_a, q_b, zacc_v, acc_sp,
              gs_a, gs_b, ss_a, ss_b):
    c = lax.axis_index("c")
    s = lax.axis_index("s")
    zeros32 = jnp.zeros((32,), jnp.bfloat16)

    def zacc_body(i, _):
        zacc_v[i, pl.ds(0, 32)] = zeros32
        return 0

    lax.fori_loop(0, 98, zacc_body, 0)

    def start_g(j, buf, sem):
        pltpu.async_copy(xt_hbm.at[src_c.at[j]], buf, sem)

    def wait_g(buf, sem):
        pltpu.make_async_copy(xt_hbm.at[src_c.at[0]], buf, sem).wait()

    def start_s(j, qbuf, sem):
        pltpu.async_copy(qbuf, acc_sp.at[dst_c.at[j]], sem, add=True)

    def wait_s(qbuf, sem):
        pltpu.make_async_copy(qbuf, acc_sp.at[dst_c.at[0]], sem).wait()

    def scale(j, buf, qbuf):
        def scale_body(g, _):
            cw = ce_c[j, pl.ds(16 * g, 16)]
            for lane in range(16):
                e = 16 * g + lane
                cv = cw[lane]
                lo = buf[e, pl.ds(0, 16)] * cv
                hi = buf[e, pl.ds(16, 16)] * cv
                qbuf[e, pl.ds(0, 32)] = plsc.pack(
                    lo, hi, format=plsc.PackFormat.INTERLEAVED)
            return 0

        lax.fori_loop(0, 8, scale_body, 0)

    def pair_body(p, _):
        r = c * 2 + lax.shift_right_logical(p, 2)
        cc = lax.bitwise_and(p, 3)
        for k in range(32):
            pltpu.sync_copy(
                zacc_v, acc_sp.at[pl.ds(s * 3136 + k * 98, 98)])
        plsc.subcore_barrier()

        for half in range(2):
            pltpu.sync_copy(
                srcp_hbm.at[r, s, pl.ds(half * HROWS, HROWS)], src_c)
            pltpu.sync_copy(
                dstp_hbm.at[r, s, pl.ds(half * HROWS, HROWS)], dst_c)
            pltpu.sync_copy(
                cep_hbm.at[r, s, pl.ds(half * HROWS, HROWS)], ce_c)
            off = cc * NX

            def shift_body(j, _):
                for k in range(8):
                    sl = pl.ds(16 * k, 16)
                    src_c[j, sl] = src_c[j, sl] + off
                return 0

            lax.fori_loop(0, HROWS, shift_body, 0)

            start_g(0, rows_a, gs_a)

            def pipe_body(i, _):
                ja = 2 * i
                jb = 2 * i + 1
                wait_g(rows_a, gs_a)

                @pl.when(i > 0)
                def _():
                    wait_s(q_b, ss_b)

                start_g(jb, rows_b, gs_b)
                scale(ja, rows_a, q_a)
                start_s(ja, q_a, ss_a)
                wait_g(rows_b, gs_b)
                wait_s(q_a, ss_a)

                @pl.when(ja + 2 < HROWS)
                def _():
                    start_g(ja + 2, rows_a, gs_a)

                scale(jb, rows_b, q_b)
                start_s(jb, q_b, ss_b)
                return 0

            lax.fori_loop(0, HROWS // 2, pipe_body, 0)
            wait_s(q_b, ss_b)

        plsc.subcore_barrier()
        pltpu.sync_copy(acc_sp.at[pl.ds(s * 3136, 3136)],
                        P_hbm.at[r, cc, pl.ds(s * 3136, 3136)])
        plsc.subcore_barrier()
        return 0

    lax.fori_loop(0, 8, pair_body, 0)


def _sc_coeffs(srcp, dstp, ewp):
    mesh = plsc.VectorSubcoreMesh(core_axis_name="c", subcore_axis_name="s")
    f = pl.kernel(
        _ce_body,
        mesh=mesh,
        compiler_params=_SC_PARAMS,
        out_type=jax.ShapeDtypeStruct((N_REL, 16, EROWS, 128), jnp.float32),
        scratch_types=[
            pltpu.VMEM((EROWS, 128), jnp.int32),
            pltpu.VMEM((EROWS, 128), jnp.int32),
            pltpu.VMEM((EROWS, 128), jnp.float32),
            pltpu.VMEM((ND,), jnp.float32),
            pltpu.VMEM((NDS,), jnp.float32),
            pltpu.VMEM((NDS,), jnp.float32),
            pltpu.VMEM_SHARED((NHIST * ND,), jnp.float32),
            pltpu.VMEM_SHARED((ND,), jnp.float32),
            pltpu.VMEM_SHARED((ND,), jnp.float32),
        ],
    )
    return f(srcp, dstp, ewp)


def _sc_aggregate(xt, srcp, dstp, cep):
    mesh = plsc.VectorSubcoreMesh(core_axis_name="c", subcore_axis_name="s")
    f = pl.kernel(
        _agg_body,
        mesh=mesh,
        compiler_params=_SC_PARAMS,
        out_type=jax.ShapeDtypeStruct((N_REL, 4, NPR, 32), jnp.bfloat16),
        scratch_types=[
            pltpu.VMEM((HROWS, 128), jnp.int32),
            pltpu.VMEM((HROWS, 128), jnp.int32),
            pltpu.VMEM((HROWS, 128), jnp.float32),
            pltpu.VMEM((128, 32), jnp.float32),
            pltpu.VMEM((128, 32), jnp.float32),
            pltpu.VMEM((128, 32), jnp.bfloat16),
            pltpu.VMEM((128, 32), jnp.bfloat16),
            pltpu.VMEM((98, 32), jnp.bfloat16),
            pltpu.VMEM_SHARED((NPR, 32), jnp.bfloat16),
            pltpu.SemaphoreType.DMA,
            pltpu.SemaphoreType.DMA,
            pltpu.SemaphoreType.DMA,
            pltpu.SemaphoreType.DMA,
        ],
    )
    return f(xt, srcp, dstp, cep)


_NB = 2000


def _mm_body(P_ref, W_ref, mb_ref, o_ref):
    acc = jnp.broadcast_to(mb_ref[0][None, :], (_NB, 128)).astype(jnp.float32)
    for r in range(4):
        pcat = jnp.concatenate(
            [P_ref[r, cch] for cch in range(4)], axis=1).astype(jnp.float32)
        acc = acc + jnp.dot(pcat, W_ref[r], preferred_element_type=jnp.float32)
    o_ref[...] = acc


def _combine(P, Ws, mb):
    return pl.pallas_call(
        _mm_body,
        grid=(N_NODES // _NB,),
        in_specs=[
            pl.BlockSpec((4, 4, _NB, 32), lambda i: (0, 0, i, 0)),
            pl.BlockSpec((4, 128, 128), lambda i: (0, 0, 0)),
            pl.BlockSpec((1, 128), lambda i: (0, 0)),
        ],
        out_specs=pl.BlockSpec((_NB, 128), lambda i: (i, 0)),
        out_shape=jax.ShapeDtypeStruct((N_NODES, 128), jnp.float32),
    )(P, Ws, mb)


def kernel(node_embedding, edge_index, edge_weight, W, b):
    x = node_embedding
    src = edge_index[:, 0, :].astype(jnp.int32)
    dst = edge_index[:, 1, :].astype(jnp.int32)

    srcp = jnp.full((N_REL, EP), N_NODES, jnp.int32).at[:, :N_EDGES].set(src)
    dstp = jnp.full((N_REL, EP), N_NODES, jnp.int32).at[:, :N_EDGES].set(dst)
    ewp = jnp.zeros((N_REL, EP), jnp.float32).at[:, :N_EDGES].set(edge_weight)
    srcp = srcp.reshape(N_REL, 16, EROWS, 128)
    dstp = dstp.reshape(N_REL, 16, EROWS, 128)
    ewp = ewp.reshape(N_REL, 16, EROWS, 128)

    xt = (jnp.zeros((4, NX, 32), jnp.float32)
          .at[:, :N_NODES].set(x.reshape(N_NODES, 4, 32).transpose(1, 0, 2))
          .reshape(4 * NX, 32))

    cep = _sc_coeffs(srcp, dstp, ewp)
    P = _sc_aggregate(xt, srcp, dstp, cep)
    mb = jnp.mean(b, axis=0, keepdims=True)
    Wp = (W.reshape(N_REL, 4, 2, 16, 128).transpose(0, 1, 3, 2, 4)
          .reshape(N_REL, 128, 128))
    return _combine(P, Wp / N_REL, mb)


# speedup vs baseline: 1.0250x; 1.0250x over previous
import jax
import jax.numpy as jnp
from jax import lax
from jax.experimental import pallas as pl
from jax.experimental.pallas import tpu as pltpu
from jax.experimental.pallas import tpu_sc as plsc

N_NODES = 50000
N_REL = 4
N_EDGES = 160000
NX = 50048
EROWS = 80
EP = 16 * EROWS * 128
ND = 50176
NHIST = 8
NDS = ND // 16
NPR = 50176
HROWS = EROWS // 2

_SC_PARAMS = pltpu.CompilerParams(
    needs_layout_passes=False, use_tc_tiling_on_sc=False)


def _rsqrt16(x):
    xi = plsc.bitcast(x, jnp.int32)
    yi = jnp.int32(0x5F3759DF) - lax.shift_right_logical(xi, 1)
    y = plsc.bitcast(yi, jnp.float32)
    for _ in range(3):
        y = y * (1.5 - 0.5 * x * y * y)
    return y


def _ce_body(srcp_hbm, dstp_hbm, ewp_hbm, cep_hbm,
             src_c, dst_c, ce_c, deg_l, redbuf, sumbuf,
             stag_sp, degsrc_sp, degdst_sp):
    c = lax.axis_index("c")
    s = lax.axis_index("s")
    zeros16 = jnp.zeros((16,), jnp.float32)
    ones16 = jnp.ones((16,), jnp.float32)

    def zero_deg_l():
        def body(i, _):
            deg_l[pl.ds(16 * i, 16)] = zeros16
            return 0
        lax.fori_loop(0, ND // 16, body, 0)

    def hist_idx():
        def body(j, _):
            for k in range(8):
                v = src_c[j, pl.ds(16 * k, 16)]
                plsc.addupdate_scatter(deg_l, [v], ones16)
            return 0
        lax.fori_loop(0, EROWS, body, 0)

    def hist_phase(idx_hbm, r):
        @pl.when(s < NHIST)
        def _():
            zero_deg_l()
            for q in range(16 // NHIST):
                pltpu.sync_copy(idx_hbm.at[r, s * (16 // NHIST) + q], src_c)
                hist_idx()
            pltpu.sync_copy(deg_l, stag_sp.at[pl.ds(s * ND, ND)])

    def reduce_phase(dsp):
        pltpu.sync_copy(stag_sp.at[pl.ds(s * NDS, NDS)], sumbuf)
        for k in range(1, NHIST):
            pltpu.sync_copy(stag_sp.at[pl.ds(k * ND + s * NDS, NDS)], redbuf)

            def add_body(i, _):
                sl = pl.ds(16 * i, 16)
                sumbuf[sl] = sumbuf[sl] + redbuf[sl]
                return 0

            lax.fori_loop(0, NDS // 16, add_body, 0)
        pltpu.sync_copy(sumbuf, dsp.at[pl.ds(s * NDS, NDS)])

    def ce_pass(idx_ref):
        def body(j, _):
            for k in range(8):
                sl = pl.ds(16 * k, 16)
                v = idx_ref[j, sl]
                d = plsc.load_gather(deg_l, [v])
                y = _rsqrt16(jnp.maximum(d, 1.0))
                ce_c[j, sl] = ce_c[j, sl] * y
            return 0
        lax.fori_loop(0, EROWS, body, 0)

    for rl in range(2):
        r = c * 2 + rl
        pltpu.sync_copy(dstp_hbm.at[r, s], dst_c)
        pltpu.sync_copy(ewp_hbm.at[r, s], ce_c)

        hist_phase(srcp_hbm, r)
        plsc.subcore_barrier()
        reduce_phase(degsrc_sp)
        plsc.subcore_barrier()
        hist_phase(dstp_hbm, r)
        plsc.subcore_barrier()
        reduce_phase(degdst_sp)
        plsc.subcore_barrier()

        pltpu.sync_copy(srcp_hbm.at[r, s], src_c)
        pltpu.sync_copy(degsrc_sp, deg_l)
        ce_pass(src_c)
        pltpu.sync_copy(degdst_sp, deg_l)
        ce_pass(dst_c)
        pltpu.sync_copy(ce_c, cep_hbm.at[r, s])
        plsc.subcore_barrier()


def _agg_body(xt_hbm, srcp_hbm, dstp_hbm, cep_hbm, P_hbm,
              src_c, dst_c, ce_c, rows_a, rows_b, zacc_v, acc_sp,
              gs_a, gs_b, ss_a, ss_b):
    c = lax.axis_index("c")
    s = lax.axis_index("s")
    zeros16 = jnp.zeros((16,), jnp.float32)

    def zacc_body(i, _):
        zacc_v[i, pl.ds(0, 16)] = zeros16
        zacc_v[i, pl.ds(16, 16)] = zeros16
        return 0

    lax.fori_loop(0, 98, zacc_body, 0)

    def start_g(j, buf, sem):
        pltpu.async_copy(xt_hbm.at[src_c.at[j]], buf, sem)

    def wait_g(buf, sem):
        pltpu.make_async_copy(xt_hbm.at[src_c.at[0]], buf, sem).wait()

    def start_s(j, qbuf, sem):
        pltpu.async_copy(qbuf, acc_sp.at[dst_c.at[j]], sem, add=True)

    def wait_s(qbuf, sem):
        pltpu.make_async_copy(qbuf, acc_sp.at[dst_c.at[0]], sem).wait()

    def scale(j, buf, qbuf):
        del qbuf
        def scale_body(g, _):
            cw = ce_c[j, pl.ds(16 * g, 16)]
            for lane in range(16):
                e = 16 * g + lane
                cv = cw[lane]
                buf[e, pl.ds(0, 16)] = buf[e, pl.ds(0, 16)] * cv
                buf[e, pl.ds(16, 16)] = buf[e, pl.ds(16, 16)] * cv
            return 0

        lax.fori_loop(0, 8, scale_body, 0)

    def pair_body(p, _):
        r = c * 2 + lax.shift_right_logical(p, 2)
        cc = lax.bitwise_and(p, 3)
        for k in range(32):
            pltpu.sync_copy(
                zacc_v, acc_sp.at[pl.ds(s * 3136 + k * 98, 98)])
        plsc.subcore_barrier()

        for half in range(2):
            pltpu.sync_copy(
                srcp_hbm.at[r, s, pl.ds(half * HROWS, HROWS)], src_c)
            pltpu.sync_copy(
                dstp_hbm.at[r, s, pl.ds(half * HROWS, HROWS)], dst_c)
            pltpu.sync_copy(
                cep_hbm.at[r, s, pl.ds(half * HROWS, HROWS)], ce_c)
            off = cc * NX

            def shift_body(j, _):
                for k in range(8):
                    sl = pl.ds(16 * k, 16)
                    src_c[j, sl] = src_c[j, sl] + off
                return 0

            lax.fori_loop(0, HROWS, shift_body, 0)

            start_g(0, rows_a, gs_a)

            def pipe_body(i, _):
                ja = 2 * i
                jb = 2 * i + 1
                wait_g(rows_a, gs_a)

                @pl.when(i > 0)
                def _():
                    wait_s(rows_b, ss_b)

                start_g(jb, rows_b, gs_b)
                scale(ja, rows_a, None)
                start_s(ja, rows_a, ss_a)
                wait_g(rows_b, gs_b)
                wait_s(rows_a, ss_a)

                @pl.when(ja + 2 < HROWS)
                def _():
                    start_g(ja + 2, rows_a, gs_a)

                scale(jb, rows_b, None)
                start_s(jb, rows_b, ss_b)
                return 0

            lax.fori_loop(0, HROWS // 2, pipe_body, 0)
            wait_s(rows_b, ss_b)

        plsc.subcore_barrier()
        pltpu.sync_copy(acc_sp.at[pl.ds(s * 3136, 3136)],
                        P_hbm.at[r, cc, pl.ds(s * 3136, 3136)])
        plsc.subcore_barrier()
        return 0

    lax.fori_loop(0, 8, pair_body, 0)


def _sc_coeffs(srcp, dstp, ewp):
    mesh = plsc.VectorSubcoreMesh(core_axis_name="c", subcore_axis_name="s")
    f = pl.kernel(
        _ce_body,
        mesh=mesh,
        compiler_params=_SC_PARAMS,
        out_type=jax.ShapeDtypeStruct((N_REL, 16, EROWS, 128), jnp.float32),
        scratch_types=[
            pltpu.VMEM((EROWS, 128), jnp.int32),
            pltpu.VMEM((EROWS, 128), jnp.int32),
            pltpu.VMEM((EROWS, 128), jnp.float32),
            pltpu.VMEM((ND,), jnp.float32),
            pltpu.VMEM((NDS,), jnp.float32),
            pltpu.VMEM((NDS,), jnp.float32),
            pltpu.VMEM_SHARED((NHIST * ND,), jnp.float32),
            pltpu.VMEM_SHARED((ND,), jnp.float32),
            pltpu.VMEM_SHARED((ND,), jnp.float32),
        ],
    )
    return f(srcp, dstp, ewp)


def _sc_aggregate(xt, srcp, dstp, cep):
    mesh = plsc.VectorSubcoreMesh(core_axis_name="c", subcore_axis_name="s")
    f = pl.kernel(
        _agg_body,
        mesh=mesh,
        compiler_params=_SC_PARAMS,
        out_type=jax.ShapeDtypeStruct((N_REL, 4, NPR, 32), jnp.float32),
        scratch_types=[
            pltpu.VMEM((HROWS, 128), jnp.int32),
            pltpu.VMEM((HROWS, 128), jnp.int32),
            pltpu.VMEM((HROWS, 128), jnp.float32),
            pltpu.VMEM((128, 32), jnp.float32),
            pltpu.VMEM((128, 32), jnp.float32),
            pltpu.VMEM((98, 32), jnp.float32),
            pltpu.VMEM_SHARED((NPR, 32), jnp.float32),
            pltpu.SemaphoreType.DMA,
            pltpu.SemaphoreType.DMA,
            pltpu.SemaphoreType.DMA,
            pltpu.SemaphoreType.DMA,
        ],
    )
    return f(xt, srcp, dstp, cep)


_NB = 2000


def _mm_body(P_ref, W_ref, mb_ref, o_ref):
    acc = jnp.broadcast_to(mb_ref[0][None, :], (_NB, 128)).astype(jnp.float32)
    for r in range(4):
        pcat = jnp.concatenate(
            [P_ref[r, cch] for cch in range(4)], axis=1).astype(jnp.float32)
        acc = acc + jnp.dot(pcat, W_ref[r], preferred_element_type=jnp.float32)
    o_ref[...] = acc


def _combine(P, Ws, mb):
    return pl.pallas_call(
        _mm_body,
        grid=(N_NODES // _NB,),
        in_specs=[
            pl.BlockSpec((4, 4, _NB, 32), lambda i: (0, 0, i, 0)),
            pl.BlockSpec((4, 128, 128), lambda i: (0, 0, 0)),
            pl.BlockSpec((1, 128), lambda i: (0, 0)),
        ],
        out_specs=pl.BlockSpec((_NB, 128), lambda i: (i, 0)),
        out_shape=jax.ShapeDtypeStruct((N_NODES, 128), jnp.float32),
    )(P, Ws, mb)


def kernel(node_embedding, edge_index, edge_weight, W, b):
    x = node_embedding
    src = edge_index[:, 0, :].astype(jnp.int32)
    dst = edge_index[:, 1, :].astype(jnp.int32)

    srcp = jnp.full((N_REL, EP), N_NODES, jnp.int32).at[:, :N_EDGES].set(src)
    dstp = jnp.full((N_REL, EP), N_NODES, jnp.int32).at[:, :N_EDGES].set(dst)
    ewp = jnp.zeros((N_REL, EP), jnp.float32).at[:, :N_EDGES].set(edge_weight)
    srcp = srcp.reshape(N_REL, 16, EROWS, 128)
    dstp = dstp.reshape(N_REL, 16, EROWS, 128)
    ewp = ewp.reshape(N_REL, 16, EROWS, 128)

    xt = (jnp.zeros((4, NX, 32), jnp.float32)
          .at[:, :N_NODES].set(x.reshape(N_NODES, 4, 32).transpose(1, 0, 2))
          .reshape(4 * NX, 32))

    cep = _sc_coeffs(srcp, dstp, ewp)
    P = _sc_aggregate(xt, srcp, dstp, cep)
    mb = jnp.mean(b, axis=0, keepdims=True)
    return _combine(P, W / N_REL, mb)


# speedup vs baseline: 1.5109x; 1.4740x over previous
import jax
import jax.numpy as jnp
from jax import lax
from jax.experimental import pallas as pl
from jax.experimental.pallas import tpu as pltpu
from jax.experimental.pallas import tpu_sc as plsc

N_NODES = 50000
N_REL = 4
N_EDGES = 160000
NX = 50048
EROWS = 80
EP = 16 * EROWS * 128
ND = 50176
NHIST = 8
NDS = ND // 16
NPR = 50176
HROWS = EROWS // 2
NCH = 2

_SC_PARAMS = pltpu.CompilerParams(
    needs_layout_passes=False, use_tc_tiling_on_sc=False)


def _rsqrt16(x):
    xi = plsc.bitcast(x, jnp.int32)
    yi = jnp.int32(0x5F3759DF) - lax.shift_right_logical(xi, 1)
    y = plsc.bitcast(yi, jnp.float32)
    for _ in range(3):
        y = y * (1.5 - 0.5 * x * y * y)
    return y


def _ce_body(srcp_hbm, dstp_hbm, ewp_hbm, cep_hbm,
             src_c, dst_c, ce_c, deg_l, redbuf, sumbuf,
             stag_sp, degsrc_sp, degdst_sp):
    c = lax.axis_index("c")
    s = lax.axis_index("s")
    zeros16 = jnp.zeros((16,), jnp.float32)
    ones16 = jnp.ones((16,), jnp.float32)

    def zero_deg_l():
        def body(i, _):
            deg_l[pl.ds(16 * i, 16)] = zeros16
            return 0
        lax.fori_loop(0, ND // 16, body, 0)

    def hist_idx():
        def body(j, _):
            for k in range(8):
                v = src_c[j, pl.ds(16 * k, 16)]
                plsc.addupdate_scatter(deg_l, [v], ones16)
            return 0
        lax.fori_loop(0, EROWS, body, 0)

    def hist_phase(idx_hbm, r):
        @pl.when(s < NHIST)
        def _():
            zero_deg_l()
            for q in range(16 // NHIST):
                pltpu.sync_copy(idx_hbm.at[r, s * (16 // NHIST) + q], src_c)
                hist_idx()
            pltpu.sync_copy(deg_l, stag_sp.at[pl.ds(s * ND, ND)])

    def reduce_phase(dsp):
        pltpu.sync_copy(stag_sp.at[pl.ds(s * NDS, NDS)], sumbuf)
        for k in range(1, NHIST):
            pltpu.sync_copy(stag_sp.at[pl.ds(k * ND + s * NDS, NDS)], redbuf)

            def add_body(i, _):
                sl = pl.ds(16 * i, 16)
                sumbuf[sl] = sumbuf[sl] + redbuf[sl]
                return 0

            lax.fori_loop(0, NDS // 16, add_body, 0)
        pltpu.sync_copy(sumbuf, dsp.at[pl.ds(s * NDS, NDS)])

    def ce_pass(idx_ref):
        def body(j, _):
            for k in range(8):
                sl = pl.ds(16 * k, 16)
                v = idx_ref[j, sl]
                d = plsc.load_gather(deg_l, [v])
                y = _rsqrt16(jnp.maximum(d, 1.0))
                ce_c[j, sl] = ce_c[j, sl] * y
            return 0
        lax.fori_loop(0, EROWS, body, 0)

    for rl in range(2):
        r = c * 2 + rl
        pltpu.sync_copy(dstp_hbm.at[r, s], dst_c)
        pltpu.sync_copy(ewp_hbm.at[r, s], ce_c)

        hist_phase(srcp_hbm, r)
        plsc.subcore_barrier()
        reduce_phase(degsrc_sp)
        plsc.subcore_barrier()
        hist_phase(dstp_hbm, r)
        plsc.subcore_barrier()
        reduce_phase(degdst_sp)
        plsc.subcore_barrier()

        pltpu.sync_copy(srcp_hbm.at[r, s], src_c)
        pltpu.sync_copy(degsrc_sp, deg_l)
        ce_pass(src_c)
        pltpu.sync_copy(degdst_sp, deg_l)
        ce_pass(dst_c)
        pltpu.sync_copy(ce_c, cep_hbm.at[r, s])
        plsc.subcore_barrier()


def _agg_body(xt_hbm, srcp_hbm, dstp_hbm, cep_hbm, P_hbm,
              src_c, dst_c, ce_c, rows_a, rows_b, zacc_v, acc_sp,
              gs_a, gs_b, ss_a, ss_b):
    c = lax.axis_index("c")
    s = lax.axis_index("s")
    zeros32 = jnp.zeros((32,), jnp.bfloat16)

    def zacc_body(i, _):
        zacc_v[i, pl.ds(0, 32)] = zeros32
        zacc_v[i, pl.ds(32, 32)] = zeros32
        return 0

    lax.fori_loop(0, 98, zacc_body, 0)

    def start_g(j, buf, sem):
        pltpu.async_copy(xt_hbm.at[src_c.at[j]], buf, sem)

    def wait_g(buf, sem):
        pltpu.make_async_copy(xt_hbm.at[src_c.at[0]], buf, sem).wait()

    def start_s(j, qbuf, sem):
        pltpu.async_copy(qbuf, acc_sp.at[dst_c.at[j]], sem, add=True)

    def wait_s(qbuf, sem):
        pltpu.make_async_copy(qbuf, acc_sp.at[dst_c.at[0]], sem).wait()

    def scale(j, buf):
        def scale_body(g, _):
            cw = ce_c[j, pl.ds(16 * g, 16)]
            for lane in range(16):
                e = 16 * g + lane
                cs = jnp.full((16,), cw[lane], jnp.float32)
                cv = plsc.pack(cs, cs, format=plsc.PackFormat.INTERLEAVED)
                buf[e, pl.ds(0, 32)] = buf[e, pl.ds(0, 32)] * cv
                buf[e, pl.ds(32, 32)] = buf[e, pl.ds(32, 32)] * cv
            return 0

        lax.fori_loop(0, 8, scale_body, 0)

    def pair_body(p, _):
        r = c * 2 + lax.shift_right_logical(p, 1)
        cc = lax.bitwise_and(p, 1)
        for k in range(32):
            pltpu.sync_copy(
                zacc_v, acc_sp.at[pl.ds(s * 3136 + k * 98, 98)])
        plsc.subcore_barrier()

        for half in range(2):
            pltpu.sync_copy(
                srcp_hbm.at[r, s, pl.ds(half * HROWS, HROWS)], src_c)
            pltpu.sync_copy(
                dstp_hbm.at[r, s, pl.ds(half * HROWS, HROWS)], dst_c)
            pltpu.sync_copy(
                cep_hbm.at[r, s, pl.ds(half * HROWS, HROWS)], ce_c)
            off = cc * NX

            def shift_body(j, _):
                for k in range(8):
                    sl = pl.ds(16 * k, 16)
                    src_c[j, sl] = src_c[j, sl] + off
                return 0

            lax.fori_loop(0, HROWS, shift_body, 0)

            start_g(0, rows_a, gs_a)

            def pipe_body(i, _):
                ja = 2 * i
                jb = 2 * i + 1
                wait_g(rows_a, gs_a)

                @pl.when(i > 0)
                def _():
                    wait_s(rows_b, ss_b)

                start_g(jb, rows_b, gs_b)
                scale(ja, rows_a)
                start_s(ja, rows_a, ss_a)
                wait_g(rows_b, gs_b)
                wait_s(rows_a, ss_a)

                @pl.when(ja + 2 < HROWS)
                def _():
                    start_g(ja + 2, rows_a, gs_a)

                scale(jb, rows_b)
                start_s(jb, rows_b, ss_b)
                return 0

            lax.fori_loop(0, HROWS // 2, pipe_body, 0)
            wait_s(rows_b, ss_b)

        plsc.subcore_barrier()
        pltpu.sync_copy(acc_sp.at[pl.ds(s * 3136, 3136)],
                        P_hbm.at[r, cc, pl.ds(s * 3136, 3136)])
        plsc.subcore_barrier()
        return 0

    lax.fori_loop(0, 2 * NCH, pair_body, 0)


def _sc_coeffs(srcp, dstp, ewp):
    mesh = plsc.VectorSubcoreMesh(core_axis_name="c", subcore_axis_name="s")
    f = pl.kernel(
        _ce_body,
        mesh=mesh,
        compiler_params=_SC_PARAMS,
        out_type=jax.ShapeDtypeStruct((N_REL, 16, EROWS, 128), jnp.float32),
        scratch_types=[
            pltpu.VMEM((EROWS, 128), jnp.int32),
            pltpu.VMEM((EROWS, 128), jnp.int32),
            pltpu.VMEM((EROWS, 128), jnp.float32),
            pltpu.VMEM((ND,), jnp.float32),
            pltpu.VMEM((NDS,), jnp.float32),
            pltpu.VMEM((NDS,), jnp.float32),
            pltpu.VMEM_SHARED((NHIST * ND,), jnp.float32),
            pltpu.VMEM_SHARED((ND,), jnp.float32),
            pltpu.VMEM_SHARED((ND,), jnp.float32),
        ],
    )
    return f(srcp, dstp, ewp)


def _sc_aggregate(xt, srcp, dstp, cep):
    mesh = plsc.VectorSubcoreMesh(core_axis_name="c", subcore_axis_name="s")
    f = pl.kernel(
        _agg_body,
        mesh=mesh,
        compiler_params=_SC_PARAMS,
        out_type=jax.ShapeDtypeStruct((N_REL, NCH, NPR, 64), jnp.bfloat16),
        scratch_types=[
            pltpu.VMEM((HROWS, 128), jnp.int32),
            pltpu.VMEM((HROWS, 128), jnp.int32),
            pltpu.VMEM((HROWS, 128), jnp.float32),
            pltpu.VMEM((128, 64), jnp.bfloat16),
            pltpu.VMEM((128, 64), jnp.bfloat16),
            pltpu.VMEM((98, 64), jnp.bfloat16),
            pltpu.VMEM_SHARED((NPR, 64), jnp.bfloat16),
            pltpu.SemaphoreType.DMA,
            pltpu.SemaphoreType.DMA,
            pltpu.SemaphoreType.DMA,
            pltpu.SemaphoreType.DMA,
        ],
    )
    return f(xt, srcp, dstp, cep)


_NB = 2000


def _mm_body(P_ref, W_ref, mb_ref, o_ref):
    acc = jnp.broadcast_to(mb_ref[0][None, :], (_NB, 128)).astype(jnp.float32)
    for r in range(4):
        pcat = jnp.concatenate(
            [P_ref[r, cch] for cch in range(NCH)], axis=1).astype(jnp.float32)
        acc = acc + jnp.dot(pcat, W_ref[r], preferred_element_type=jnp.float32)
    o_ref[...] = acc


def _combine(P, Ws, mb):
    return pl.pallas_call(
        _mm_body,
        grid=(N_NODES // _NB,),
        in_specs=[
            pl.BlockSpec((4, NCH, _NB, 64), lambda i: (0, 0, i, 0)),
            pl.BlockSpec((4, 128, 128), lambda i: (0, 0, 0)),
            pl.BlockSpec((1, 128), lambda i: (0, 0)),
        ],
        out_specs=pl.BlockSpec((_NB, 128), lambda i: (i, 0)),
        out_shape=jax.ShapeDtypeStruct((N_NODES, 128), jnp.float32),
    )(P, Ws, mb)


def kernel(node_embedding, edge_index, edge_weight, W, b):
    x = node_embedding
    src = edge_index[:, 0, :].astype(jnp.int32)
    dst = edge_index[:, 1, :].astype(jnp.int32)

    srcp = jnp.full((N_REL, EP), N_NODES, jnp.int32).at[:, :N_EDGES].set(src)
    dstp = jnp.full((N_REL, EP), N_NODES, jnp.int32).at[:, :N_EDGES].set(dst)
    ewp = jnp.zeros((N_REL, EP), jnp.float32).at[:, :N_EDGES].set(edge_weight)
    srcp = srcp.reshape(N_REL, 16, EROWS, 128)
    dstp = dstp.reshape(N_REL, 16, EROWS, 128)
    ewp = ewp.reshape(N_REL, 16, EROWS, 128)

    xt = (jnp.zeros((NCH, NX, 64), jnp.bfloat16)
          .at[:, :N_NODES].set(
              x.reshape(N_NODES, NCH, 64).transpose(1, 0, 2)
              .astype(jnp.bfloat16))
          .reshape(NCH * NX, 64))

    cep = _sc_coeffs(srcp, dstp, ewp)
    P = _sc_aggregate(xt, srcp, dstp, cep)
    mb = jnp.mean(b, axis=0, keepdims=True)
    return _combine(P, W / N_REL, mb)
